# baseline (speedup 1.0000x reference)
import functools

import jax
import jax.numpy as jnp
from jax import lax
from jax.experimental import pallas as pl
from jax.experimental.pallas import tpu as pltpu
from jax.experimental.pallas import tpu_sc as plsc

N_VOCAB = 1000000
D_MODEL = 64
BATCH = 4096
SEQ = 200

_NC = 2
_NS = 16
_NW = _NC * _NS

_B = BATCH * SEQ
_CH = 128
_BPW = _B // _NW
_NCHUNK = _BPW // _CH

_VB = 2048
_NVB = 245
_K = _VB * _NVB
_MAXB = (N_VOCAB - 1) // _VB


def _transpose_body(a_ref, b_ref, out_ref):
    out_ref[:, 0:D_MODEL] = a_ref[...].T
    out_ref[:, D_MODEL:2 * D_MODEL] = b_ref[...].T


def _build_table(w):
    return pl.pallas_call(
        _transpose_body,
        grid=(_NVB,),
        in_specs=[
            pl.BlockSpec((D_MODEL, _VB), lambda i: (0, i)),
            pl.BlockSpec((D_MODEL, _VB),
                         lambda i: (0, jnp.minimum(i + _NVB, _MAXB))),
        ],
        out_specs=pl.BlockSpec((_VB, 2 * D_MODEL), lambda i: (i, 0)),
        out_shape=jax.ShapeDtypeStruct((_K, 2 * D_MODEL), jnp.float32),
    )(w, w)


@functools.partial(
    pl.kernel,
    out_type=jax.ShapeDtypeStruct((_B // 2, 2 * D_MODEL), jnp.float32),
    mesh=plsc.VectorSubcoreMesh(core_axis_name="c", subcore_axis_name="s"),
    scratch_types=[
        pltpu.VMEM((_NCHUNK, _CH), jnp.int32),
        pltpu.VMEM((_CH,), jnp.int32),
        pltpu.VMEM((_CH,), jnp.int32),
        pltpu.VMEM((_CH, 2 * D_MODEL), jnp.float32),
        pltpu.VMEM((_CH // 2, 2 * D_MODEL), jnp.float32),
        pltpu.SemaphoreType.DMA,
    ],
)
def _sc_gather(table_hbm, idx_hbm, out_hbm, idx_v, ridx_v, offs_v, rows_v,
               out_v, sem):
    wid = lax.axis_index("s") * _NC + lax.axis_index("c")
    base = wid * (_BPW // 2)
    pltpu.sync_copy(idx_hbm.at[wid], idx_v)

    def chunk(c, _):
        for g in range(_CH // 16):
            xv = idx_v[c, pl.ds(g * 16, 16)]
            hi = xv >= _K
            ridx_v[pl.ds(g * 16, 16)] = jnp.where(hi, xv - _K, xv)
            offs_v[pl.ds(g * 16, 16)] = jnp.where(hi, D_MODEL, 0)
        pltpu.async_copy(table_hbm.at[ridx_v], rows_v, sem).wait()

        def pick(g, _):
            offv = offs_v[pl.ds(g * 16, 16)]
            for k in range(16):
                i = g * 16 + k
                off = offv[k]
                for j in range(D_MODEL // 16):
                    out_v[g * 8 + k // 2,
                          pl.ds((k % 2) * D_MODEL + j * 16, 16)] = (
                        rows_v[i, pl.ds(off + j * 16, 16)])
            return _

        lax.fori_loop(0, _CH // 16, pick, None)
        pltpu.sync_copy(out_v, out_hbm.at[pl.ds(base + c * (_CH // 2),
                                                _CH // 2)])
        return _

    lax.fori_loop(0, _NCHUNK, chunk, None)


def kernel(x, W_embed):
    table = _build_table(W_embed)
    idx = x.astype(jnp.int32).reshape(_NW, _NCHUNK, _CH)
    out = _sc_gather(table, idx)
    return out.reshape(BATCH, SEQ, D_MODEL)


# speedup vs baseline: 1.4020x; 1.4020x over previous
import functools

import jax
import jax.numpy as jnp
from jax import lax
from jax.experimental import pallas as pl
from jax.experimental.pallas import tpu as pltpu
from jax.experimental.pallas import tpu_sc as plsc

N_VOCAB = 1000000
D_MODEL = 64
BATCH = 4096
SEQ = 200

_NC = 2
_NS = 16
_NW = _NC * _NS

_B = BATCH * SEQ
_CH = 128
_BPW = _B // _NW
_NCHUNK = _BPW // _CH

_VB = 2048
_NVB = 245
_K = _VB * _NVB
_MAXB = (N_VOCAB - 1) // _VB


def _transpose_body(a_ref, b_ref, out_ref):
    out_ref[:, 0:D_MODEL] = a_ref[...].T
    out_ref[:, D_MODEL:2 * D_MODEL] = b_ref[...].T


def _build_table(w):
    return pl.pallas_call(
        _transpose_body,
        grid=(_NVB,),
        in_specs=[
            pl.BlockSpec((D_MODEL, _VB), lambda i: (0, i)),
            pl.BlockSpec((D_MODEL, _VB),
                         lambda i: (0, jnp.minimum(i + _NVB, _MAXB))),
        ],
        out_specs=pl.BlockSpec((_VB, 2 * D_MODEL), lambda i: (i, 0)),
        out_shape=jax.ShapeDtypeStruct((_K, 2 * D_MODEL), jnp.float32),
    )(w, w)


@functools.partial(
    pl.kernel,
    out_type=jax.ShapeDtypeStruct((_B, D_MODEL), jnp.float32),
    mesh=plsc.VectorSubcoreMesh(core_axis_name="c", subcore_axis_name="s"),
    compiler_params=pltpu.CompilerParams(use_tc_tiling_on_sc=False),
    scratch_types=[
        pltpu.VMEM((_NCHUNK, _CH), jnp.int32),
        pltpu.VMEM((_CH,), jnp.int32),
        pltpu.VMEM((_CH, D_MODEL), jnp.float32),
        pltpu.SemaphoreType.DMA,
    ],
)
def _sc_gather(table_hbm, idx_hbm, out_hbm, idx_v, ridx_v, out_v, sem):
    wid = lax.axis_index("s") * _NC + lax.axis_index("c")
    base = wid * _BPW
    pltpu.sync_copy(idx_hbm.at[wid], idx_v)

    def chunk(c, _):
        for g in range(_CH // 16):
            xv = idx_v[c, pl.ds(g * 16, 16)]
            hi = xv >= _K
            ridx_v[pl.ds(g * 16, 16)] = jnp.where(
                hi, 2 * (xv - _K) + 1, 2 * xv)
        pltpu.async_copy(table_hbm.at[ridx_v], out_v, sem).wait()
        pltpu.sync_copy(out_v, out_hbm.at[pl.ds(base + c * _CH, _CH)])
        return _

    lax.fori_loop(0, _NCHUNK, chunk, None)


def kernel(x, W_embed):
    table = _build_table(W_embed).reshape(2 * _K, D_MODEL)
    idx = x.astype(jnp.int32).reshape(_NW, _NCHUNK, _CH)
    out = _sc_gather(table, idx)
    return out.reshape(BATCH, SEQ, D_MODEL)
